# baseline (device time: 104470 ns/iter reference)
import jax
import jax.numpy as jnp
from jax import lax
from jax.experimental import pallas as pl
from jax.experimental.pallas import tpu as pltpu

N_DEV = 4


def kernel(O, Wo):
    b, s, h, d = O.shape
    k = h * d
    n = Wo.shape[1]
    nh = n // 2
    s_chunk = s // N_DEV
    o = O.reshape(b, s, k)

    def body(o_ref, wo_ref, out_ref, ob_ref, wob_ref,
             cw_ref, ccw_ref, pa_ref, pb_ref,
             cw_ssems, cw_rsems, ccw_ssems, ccw_rsems):
        my = lax.axis_index("i")
        left = lax.rem(my + N_DEV - 1, N_DEV)
        right = lax.rem(my + 1, N_DEV)

        for bb in range(b):
            ob_ref[bb] = o_ref[bb].astype(jnp.bfloat16)
        wob_ref[...] = wo_ref[...].astype(jnp.bfloat16)

        barrier_sem = pltpu.get_barrier_semaphore()
        for nbr in (left, right):
            pl.semaphore_signal(
                barrier_sem, inc=1,
                device_id=(nbr,), device_id_type=pl.DeviceIdType.MESH,
            )
        pl.semaphore_wait(barrier_sem, 2)

        def pslab(dst, c, col0, dtype):
            row0 = c * s_chunk
            for bb in range(b):
                dst[bb] = jnp.dot(
                    ob_ref[bb, pl.ds(row0, s_chunk), :],
                    wob_ref[:, col0:col0 + nh],
                    preferred_element_type=jnp.float32,
                ).astype(dtype)

        def mk(dir_ref, ssems, rsems, t, dev):
            return pltpu.make_async_remote_copy(
                src_ref=dir_ref.at[t],
                dst_ref=dir_ref.at[t + 1],
                send_sem=ssems.at[t],
                recv_sem=rsems.at[t],
                device_id=(dev,),
                device_id_type=pl.DeviceIdType.MESH,
            )

        pslab(cw_ref.at[0], left, 0, jnp.bfloat16)
        mk(cw_ref, cw_ssems, cw_rsems, 0, right).start()
        pslab(ccw_ref.at[0], right, nh, jnp.bfloat16)
        mk(ccw_ref, ccw_ssems, ccw_rsems, 0, left).start()

        for t in range(N_DEV - 1):
            c_cw = lax.rem(my + (N_DEV + 2 - t), N_DEV)
            c_ccw = lax.rem(my + 2 + t, N_DEV)
            pslab(pa_ref, c_cw, 0, jnp.float32)
            pslab(pb_ref, c_ccw, nh, jnp.float32)
            mk(cw_ref, cw_ssems, cw_rsems, t, right).wait_recv()
            if t < N_DEV - 2:
                cw_ref[t + 1] = (
                    cw_ref[t + 1] + pa_ref[...]
                ).astype(jnp.bfloat16)
                mk(cw_ref, cw_ssems, cw_rsems, t + 1, right).start()
            else:
                out_ref[:, :, 0:nh] = cw_ref[t + 1] + pa_ref[...]
            mk(ccw_ref, ccw_ssems, ccw_rsems, t, left).wait_recv()
            if t < N_DEV - 2:
                ccw_ref[t + 1] = (
                    ccw_ref[t + 1] + pb_ref[...]
                ).astype(jnp.bfloat16)
                mk(ccw_ref, ccw_ssems, ccw_rsems, t + 1, left).start()
            else:
                out_ref[:, :, nh:n] = ccw_ref[t + 1] + pb_ref[...]

        for t in range(N_DEV - 1):
            mk(cw_ref, cw_ssems, cw_rsems, t, right).wait_send()
            mk(ccw_ref, ccw_ssems, ccw_rsems, t, left).wait_send()

    out_shape = jax.ShapeDtypeStruct((b, s_chunk, n), jnp.float32)
    return pl.pallas_call(
        body,
        out_shape=out_shape,
        in_specs=[
            pl.BlockSpec(memory_space=pltpu.VMEM),
            pl.BlockSpec(memory_space=pltpu.VMEM),
        ],
        out_specs=pl.BlockSpec(memory_space=pltpu.VMEM),
        scratch_shapes=[
            pltpu.VMEM((b, s, k), jnp.bfloat16),
            pltpu.VMEM((k, n), jnp.bfloat16),
            pltpu.VMEM((N_DEV, b, s_chunk, nh), jnp.bfloat16),
            pltpu.VMEM((N_DEV, b, s_chunk, nh), jnp.bfloat16),
            pltpu.VMEM((b, s_chunk, nh), jnp.float32),
            pltpu.VMEM((b, s_chunk, nh), jnp.float32),
            pltpu.SemaphoreType.DMA((N_DEV - 1,)),
            pltpu.SemaphoreType.DMA((N_DEV - 1,)),
            pltpu.SemaphoreType.DMA((N_DEV - 1,)),
            pltpu.SemaphoreType.DMA((N_DEV - 1,)),
        ],
        compiler_params=pltpu.CompilerParams(
            collective_id=0,
            vmem_limit_bytes=100 * 1024 * 1024,
        ),
    )(o, Wo)


# device time: 97761 ns/iter; 1.0686x vs baseline; 1.0686x over previous
import jax
import jax.numpy as jnp
from jax import lax
from jax.experimental import pallas as pl
from jax.experimental.pallas import tpu as pltpu

N_DEV = 4
N_SUB = 4


def kernel(O, Wo):
    b, s, h, d = O.shape
    k = h * d
    n = Wo.shape[1]
    nh = n // 2
    s_chunk = s // N_DEV
    bsub = b // N_SUB
    o = O.reshape(b, s, k)

    def body(o_ref, wo_ref, out_ref, ob_ref, wob_ref,
             cw_ref, ccw_ref, pa_ref, pb_ref,
             cw_ssems, cw_rsems, ccw_ssems, ccw_rsems):
        my = lax.axis_index("i")
        left = lax.rem(my + N_DEV - 1, N_DEV)
        right = lax.rem(my + 1, N_DEV)

        for bb in range(b):
            ob_ref[bb] = o_ref[bb].astype(jnp.bfloat16)
        wob_ref[...] = wo_ref[...].astype(jnp.bfloat16)

        barrier_sem = pltpu.get_barrier_semaphore()
        for nbr in (left, right):
            pl.semaphore_signal(
                barrier_sem, inc=1,
                device_id=(nbr,), device_id_type=pl.DeviceIdType.MESH,
            )
        pl.semaphore_wait(barrier_sem, 2)

        def pslab(dst, c, col0, b0, nb, dtype):
            row0 = c * s_chunk
            for bb in range(nb):
                dst[bb] = jnp.dot(
                    ob_ref[b0 + bb, pl.ds(row0, s_chunk), :],
                    wob_ref[:, col0:col0 + nh],
                    preferred_element_type=jnp.float32,
                ).astype(dtype)

        def mk(dir_ref, ssems, rsems, t, j, dev):
            sl = pl.ds(j * bsub, bsub)
            return pltpu.make_async_remote_copy(
                src_ref=dir_ref.at[t, sl],
                dst_ref=dir_ref.at[t + 1, sl],
                send_sem=ssems.at[t, j],
                recv_sem=rsems.at[t, j],
                device_id=(dev,),
                device_id_type=pl.DeviceIdType.MESH,
            )

        for j in range(N_SUB):
            sl = pl.ds(j * bsub, bsub)
            pslab(cw_ref.at[0, sl], left, 0, j * bsub, bsub, jnp.bfloat16)
            mk(cw_ref, cw_ssems, cw_rsems, 0, j, right).start()
            pslab(ccw_ref.at[0, sl], right, nh, j * bsub, bsub, jnp.bfloat16)
            mk(ccw_ref, ccw_ssems, ccw_rsems, 0, j, left).start()

        for t in range(N_DEV - 1):
            c_cw = lax.rem(my + (N_DEV + 2 - t), N_DEV)
            c_ccw = lax.rem(my + 2 + t, N_DEV)
            pslab(pa_ref, c_cw, 0, 0, b, jnp.float32)
            pslab(pb_ref, c_ccw, nh, 0, b, jnp.float32)
            for j in range(N_SUB):
                sl = pl.ds(j * bsub, bsub)
                mk(cw_ref, cw_ssems, cw_rsems, t, j, right).wait_recv()
                if t < N_DEV - 2:
                    cw_ref[t + 1, sl] = (
                        cw_ref[t + 1, sl] + pa_ref[sl]
                    ).astype(jnp.bfloat16)
                    mk(cw_ref, cw_ssems, cw_rsems, t + 1, j, right).start()
                else:
                    out_ref[sl, :, 0:nh] = cw_ref[t + 1, sl] + pa_ref[sl]
                mk(ccw_ref, ccw_ssems, ccw_rsems, t, j, left).wait_recv()
                if t < N_DEV - 2:
                    ccw_ref[t + 1, sl] = (
                        ccw_ref[t + 1, sl] + pb_ref[sl]
                    ).astype(jnp.bfloat16)
                    mk(ccw_ref, ccw_ssems, ccw_rsems, t + 1, j, left).start()
                else:
                    out_ref[sl, :, nh:n] = ccw_ref[t + 1, sl] + pb_ref[sl]

        for t in range(N_DEV - 1):
            for j in range(N_SUB):
                mk(cw_ref, cw_ssems, cw_rsems, t, j, right).wait_send()
                mk(ccw_ref, ccw_ssems, ccw_rsems, t, j, left).wait_send()

    out_shape = jax.ShapeDtypeStruct((b, s_chunk, n), jnp.float32)
    return pl.pallas_call(
        body,
        out_shape=out_shape,
        in_specs=[
            pl.BlockSpec(memory_space=pltpu.VMEM),
            pl.BlockSpec(memory_space=pltpu.VMEM),
        ],
        out_specs=pl.BlockSpec(memory_space=pltpu.VMEM),
        scratch_shapes=[
            pltpu.VMEM((b, s, k), jnp.bfloat16),
            pltpu.VMEM((k, n), jnp.bfloat16),
            pltpu.VMEM((N_DEV, b, s_chunk, nh), jnp.bfloat16),
            pltpu.VMEM((N_DEV, b, s_chunk, nh), jnp.bfloat16),
            pltpu.VMEM((b, s_chunk, nh), jnp.float32),
            pltpu.VMEM((b, s_chunk, nh), jnp.float32),
            pltpu.SemaphoreType.DMA((N_DEV - 1, N_SUB)),
            pltpu.SemaphoreType.DMA((N_DEV - 1, N_SUB)),
            pltpu.SemaphoreType.DMA((N_DEV - 1, N_SUB)),
            pltpu.SemaphoreType.DMA((N_DEV - 1, N_SUB)),
        ],
        compiler_params=pltpu.CompilerParams(
            collective_id=0,
            vmem_limit_bytes=100 * 1024 * 1024,
        ),
    )(o, Wo)
